# Initial kernel scaffold; baseline (speedup 1.0000x reference)
import functools

import jax
import jax.numpy as jnp
from jax import lax
from jax.experimental import pallas as pl
from jax.experimental.pallas import tpu as pltpu
from jax.experimental.pallas import tpu_sc as plsc

_DIM = 32
_NC = 2
_NS = 16
_NW = _NC * _NS
_CHUNK = 1024
_IDX_SUB = 128


@functools.lru_cache(maxsize=None)
def _make_gather(B: int):
    assert B % (_NW * _CHUNK) == 0
    b_per_w = B // _NW
    n_chunks = b_per_w // _CHUNK
    n_sub = _CHUNK // _IDX_SUB
    mesh = plsc.VectorSubcoreMesh(core_axis_name="c", subcore_axis_name="s")

    @functools.partial(
        pl.kernel,
        mesh=mesh,
        out_type=jax.ShapeDtypeStruct((B, _DIM), jnp.float32),
        scratch_types=[
            pltpu.VMEM((b_per_w,), jnp.int32),
            pltpu.VMEM((_CHUNK, _DIM), jnp.float32),
            pltpu.SemaphoreType.DMA,
        ],
    )
    def _k(table_hbm, idx_hbm, out_hbm, idx_v, rows_v, sem):
        wid = lax.axis_index("s") * _NC + lax.axis_index("c")
        base = wid * b_per_w
        pltpu.sync_copy(idx_hbm.at[pl.ds(base, b_per_w)], idx_v)

        def chunk_body(i, carry):
            off = i * _CHUNK
            handles = []
            for j in range(n_sub):
                handles.append(pltpu.async_copy(
                    table_hbm.at[idx_v.at[pl.ds(off + j * _IDX_SUB, _IDX_SUB)]],
                    rows_v.at[pl.ds(j * _IDX_SUB, _IDX_SUB)],
                    sem,
                ))
            for h in handles:
                h.wait()
            pltpu.sync_copy(rows_v, out_hbm.at[pl.ds(base + off, _CHUNK)])
            return carry

        lax.fori_loop(0, n_chunks, chunk_body, 0)

    return _k


def kernel(x, table):
    idx = x.reshape(-1)
    out_flat = _make_gather(idx.shape[0])(table, idx)
    return out_flat.reshape(x.shape + (_DIM,))


# speedup vs baseline: 1.1026x; 1.1026x over previous
import functools

import jax
import jax.numpy as jnp
from jax import lax
from jax.experimental import pallas as pl
from jax.experimental.pallas import tpu as pltpu
from jax.experimental.pallas import tpu_sc as plsc

_DIM = 32
_NC = 2
_NS = 16
_NW = _NC * _NS
_CHUNK = 1024
_IDX_SUB = 128


@functools.lru_cache(maxsize=None)
def _make_gather(B: int):
    assert B % (_NW * _CHUNK) == 0
    b_per_w = B // _NW
    n_chunks = b_per_w // _CHUNK
    n_sub = _CHUNK // _IDX_SUB
    mesh = plsc.VectorSubcoreMesh(core_axis_name="c", subcore_axis_name="s")

    @functools.partial(
        pl.kernel,
        mesh=mesh,
        out_type=jax.ShapeDtypeStruct((B, _DIM), jnp.float32),
        scratch_types=[
            pltpu.VMEM((b_per_w,), jnp.int32),
            pltpu.VMEM((_CHUNK, _DIM), jnp.float32),
            pltpu.SemaphoreType.DMA,
        ],
        compiler_params=pltpu.CompilerParams(use_tc_tiling_on_sc=False),
    )
    def _k(table_hbm, idx_hbm, out_hbm, idx_v, rows_v, sem):
        wid = lax.axis_index("s") * _NC + lax.axis_index("c")
        base = wid * b_per_w
        pltpu.sync_copy(idx_hbm.at[pl.ds(base, b_per_w)], idx_v)

        def chunk_body(i, carry):
            off = i * _CHUNK
            handles = []
            for j in range(n_sub):
                handles.append(pltpu.async_copy(
                    table_hbm.at[idx_v.at[pl.ds(off + j * _IDX_SUB, _IDX_SUB)]],
                    rows_v.at[pl.ds(j * _IDX_SUB, _IDX_SUB)],
                    sem,
                ))
            for h in handles:
                h.wait()
            pltpu.sync_copy(rows_v, out_hbm.at[pl.ds(base + off, _CHUNK)])
            return carry

        lax.fori_loop(0, n_chunks, chunk_body, 0)

    return _k


def kernel(x, table):
    idx = x.reshape(-1)
    out_flat = _make_gather(idx.shape[0])(table, idx)
    return out_flat.reshape(x.shape + (_DIM,))
